# baseline (speedup 1.0000x reference)
import functools

import jax
import jax.numpy as jnp
from jax import lax
from jax.experimental import pallas as pl
from jax.experimental.pallas import tpu as pltpu
from jax.experimental.pallas import tpu_sc as plsc

N_USERS = 8192
N_ITEMS = 2048
EMBED = 64
TOPK = 10
NNZ = 2 * TOPK
LAMBDA = 0.9
N_TOTAL = N_USERS + N_ITEMS

ROWB = 512
UIB = 512
N_ITEM_BLOCKS = N_ITEMS // ROWB
N_UI_BLOCKS = N_TOTAL // UIB

SC_NC = 2
SC_NS = 16
SC_LANES = 16
SC_NW = SC_NC * SC_NS
SC_ROWS = N_ITEMS // SC_NW


def _feats_body(img_ref, wi_ref, bi_ref, txt_ref, wt_ref, bt_ref,
                xi_ref, xt_ref):
    fi = jnp.dot(img_ref[...], wi_ref[...],
                 preferred_element_type=jnp.float32) + bi_ref[...]
    ni = jnp.sqrt(jnp.sum(fi * fi, axis=1, keepdims=True))
    xi_ref[...] = fi / ni
    ft = jnp.dot(txt_ref[...], wt_ref[...],
                 preferred_element_type=jnp.float32) + bt_ref[...]
    nt = jnp.sqrt(jnp.sum(ft * ft, axis=1, keepdims=True))
    xt_ref[...] = ft / nt


def _topk_collect(sim, w, iota):
    BIG = jnp.float32(3.0e4)
    vals, inds = [], []
    rowsum = jnp.zeros((ROWB, 1), jnp.float32)
    for _ in range(TOPK):
        m = jnp.max(sim, axis=1, keepdims=True)
        eq = sim == m
        idx = jnp.min(jnp.where(eq, iota, BIG), axis=1, keepdims=True)
        sel = iota == idx
        vals.append(w * m)
        inds.append(idx)
        rowsum = rowsum + w * m
        sim = jnp.where(sel, -jnp.inf, sim)
    ii = jnp.minimum(jnp.concatenate(inds, axis=1),
                     jnp.float32(N_ITEMS - 1)).astype(jnp.int32)
    return jnp.concatenate(vals, axis=1), ii, rowsum


def _knn_body(w_ref, xi_ref, xit_ref, xt_ref, xtt_ref,
              cv_ref, ix_ref, rs_ref):
    pid = pl.program_id(0)
    rows = pl.ds(pid * ROWB, ROWB)
    iota = jax.lax.broadcasted_iota(
        jnp.int32, (ROWB, N_ITEMS), 1).astype(jnp.float32)
    sim_i = jnp.dot(xi_ref[rows, :], xit_ref[...],
                    preferred_element_type=jnp.float32)
    v0, i0, rs0 = _topk_collect(sim_i, w_ref[0], iota)
    sim_t = jnp.dot(xt_ref[rows, :], xtt_ref[...],
                    preferred_element_type=jnp.float32)
    v1, i1, rs1 = _topk_collect(sim_t, w_ref[1], iota)
    cv_ref[...] = jnp.concatenate([v0, v1], axis=1)
    ix_ref[...] = jnp.concatenate([i0, i1], axis=1)
    rs_ref[...] = rs0 + rs1


def _h_body(w_ref, rs_ref, cv_ref, oi_ref, ot_ref, ie_ref,
            hd_ref, emd_ref, cvs_ref):
    pid = pl.program_id(0)
    w0 = w_ref[0]
    w1 = w_ref[1]
    rsq_blk = jax.lax.rsqrt(rs_ref[pl.ds(pid * ROWB, ROWB), :])
    d_blk = jnp.where(jnp.isinf(rsq_blk), 0.0, rsq_blk)
    emd_ref[...] = d_blk * ie_ref[pl.ds(pid * ROWB, ROWB), :]
    cvs_ref[...] = (1.0 - LAMBDA) * d_blk * cv_ref[...]
    orig = w0 * oi_ref[...] + w1 * ot_ref[...]
    hd_ref[...] = LAMBDA * jnp.dot(orig, ie_ref[...],
                                   preferred_element_type=jnp.float32)


def _sc_learned_body(emd_hbm, ix_hbm, cvs_hbm, out_hbm,
                     ix_v, cvs_v, buf_a, buf_b, out_v, sem_a, sem_b):
    wid = lax.axis_index("s") * SC_NC + lax.axis_index("c")
    base = wid * SC_ROWS
    pltpu.sync_copy(ix_hbm.at[pl.ds(base, SC_ROWS)], ix_v)
    pltpu.sync_copy(cvs_hbm.at[pl.ds(base, SC_ROWS)], cvs_v)

    zero16 = jax.lax.iota(jnp.int32, 16) * 0

    def accum(r, buf):
        for c in range(EMBED // SC_LANES):
            acc = jnp.zeros((SC_LANES,), jnp.float32)
            for k in range(NNZ):
                cvb = plsc.load_gather(cvs_v, [zero16 + r, zero16 + k])
                acc = acc + cvb * buf[k, pl.ds(c * SC_LANES, SC_LANES)]
            out_v[r, pl.ds(c * SC_LANES, SC_LANES)] = acc

    cp0 = pltpu.async_copy(emd_hbm.at[ix_v.at[0]], buf_a, sem_a)

    def body(i, _):
        r = 2 * i
        cpb = pltpu.async_copy(emd_hbm.at[ix_v.at[r + 1]], buf_b, sem_b)
        pltpu.make_async_copy(emd_hbm.at[ix_v.at[r]], buf_a, sem_a).wait()
        accum(r, buf_a)
        cpa = pltpu.async_copy(emd_hbm.at[ix_v.at[(r + 2) % SC_ROWS]],
                               buf_a, sem_a)
        pltpu.make_async_copy(emd_hbm.at[ix_v.at[r + 1]], buf_b, sem_b).wait()
        accum(r + 1, buf_b)
        return 0

    lax.fori_loop(0, SC_ROWS // 2, body, 0)
    pltpu.make_async_copy(emd_hbm.at[ix_v.at[0]], buf_a, sem_a).wait()
    pltpu.sync_copy(out_v, out_hbm.at[pl.ds(base, SC_ROWS)])


@functools.partial(jax.jit, static_argnums=())
def _sc_learned(emd, ix, cvs):
    return pl.kernel(
        _sc_learned_body,
        out_type=jax.ShapeDtypeStruct((N_ITEMS, EMBED), jnp.float32),
        mesh=plsc.VectorSubcoreMesh(core_axis_name="c", subcore_axis_name="s"),
        compiler_params=pltpu.CompilerParams(needs_layout_passes=False,
                                             use_tc_tiling_on_sc=False),
        scratch_types=[
            pltpu.VMEM((SC_ROWS, NNZ), jnp.int32),
            pltpu.VMEM((SC_ROWS, NNZ), jnp.float32),
            pltpu.VMEM((NNZ, EMBED), jnp.float32),
            pltpu.VMEM((NNZ, EMBED), jnp.float32),
            pltpu.VMEM((SC_ROWS, EMBED), jnp.float32),
            pltpu.SemaphoreType.DMA,
            pltpu.SemaphoreType.DMA,
        ],
    )(emd, ix, cvs)


def _prop1_body(adj_ref, ego_ref, e1_ref):
    e1_ref[...] = jnp.dot(adj_ref[...], ego_ref[...],
                          preferred_element_type=jnp.float32)


def _prop2_body(adj_ref, ego_ref, e1_ref, hd_ref, g_ref, out_ref):
    pid = pl.program_id(0)
    rows = pl.ds(pid * UIB, UIB)
    e2 = jnp.dot(adj_ref[...], e1_ref[...], preferred_element_type=jnp.float32)
    acc = (ego_ref[rows, :] + e1_ref[rows, :] + e2) * (1.0 / 3.0)

    @pl.when(pid >= N_USERS // UIB)
    def _():
        irows = pl.ds((pid - N_USERS // UIB) * UIB, UIB)
        h = hd_ref[irows, :] + g_ref[irows, :]
        nrm = jnp.sqrt(jnp.sum(h * h, axis=1, keepdims=True))
        out_ref[...] = acc + h / jnp.maximum(nrm, 1e-12)

    @pl.when(pid < N_USERS // UIB)
    def _():
        out_ref[...] = acc


def kernel(adj, user_emb, item_emb, image_emb, text_emb, W_img, b_img,
           W_txt, b_txt, modal_weight, image_original_adj, text_original_adj):
    w = jax.nn.softmax(modal_weight, axis=0)

    xn_img, xn_txt = pl.pallas_call(
        _feats_body,
        out_shape=(jax.ShapeDtypeStruct((N_ITEMS, EMBED), jnp.float32),
                   jax.ShapeDtypeStruct((N_ITEMS, EMBED), jnp.float32)),
    )(image_emb, W_img.astype(jnp.float32), b_img.reshape(1, EMBED),
      text_emb, W_txt.astype(jnp.float32), b_txt.reshape(1, EMBED))

    xn_img_t = xn_img.T
    xn_txt_t = xn_txt.T

    cvals, idxs, rowsum = pl.pallas_call(
        _knn_body,
        grid=(N_ITEM_BLOCKS,),
        in_specs=[
            pl.BlockSpec(memory_space=pltpu.SMEM),
            pl.BlockSpec((N_ITEMS, EMBED), lambda i: (0, 0)),
            pl.BlockSpec((EMBED, N_ITEMS), lambda i: (0, 0)),
            pl.BlockSpec((N_ITEMS, EMBED), lambda i: (0, 0)),
            pl.BlockSpec((EMBED, N_ITEMS), lambda i: (0, 0)),
        ],
        out_specs=(pl.BlockSpec((ROWB, NNZ), lambda i: (i, 0)),
                   pl.BlockSpec((ROWB, NNZ), lambda i: (i, 0)),
                   pl.BlockSpec((ROWB, 1), lambda i: (i, 0))),
        out_shape=(jax.ShapeDtypeStruct((N_ITEMS, NNZ), jnp.float32),
                   jax.ShapeDtypeStruct((N_ITEMS, NNZ), jnp.int32),
                   jax.ShapeDtypeStruct((N_ITEMS, 1), jnp.float32)),
    )(w, xn_img, xn_img_t, xn_txt, xn_txt_t)

    h_dense, em_d, cvs = pl.pallas_call(
        _h_body,
        grid=(N_ITEM_BLOCKS,),
        in_specs=[
            pl.BlockSpec(memory_space=pltpu.SMEM),
            pl.BlockSpec((N_ITEMS, 1), lambda i: (0, 0)),
            pl.BlockSpec((ROWB, NNZ), lambda i: (i, 0)),
            pl.BlockSpec((ROWB, N_ITEMS), lambda i: (i, 0)),
            pl.BlockSpec((ROWB, N_ITEMS), lambda i: (i, 0)),
            pl.BlockSpec((N_ITEMS, EMBED), lambda i: (0, 0)),
        ],
        out_specs=(pl.BlockSpec((ROWB, EMBED), lambda i: (i, 0)),
                   pl.BlockSpec((ROWB, EMBED), lambda i: (i, 0)),
                   pl.BlockSpec((ROWB, NNZ), lambda i: (i, 0))),
        out_shape=(jax.ShapeDtypeStruct((N_ITEMS, EMBED), jnp.float32),
                   jax.ShapeDtypeStruct((N_ITEMS, EMBED), jnp.float32),
                   jax.ShapeDtypeStruct((N_ITEMS, NNZ), jnp.float32)),
    )(w, rowsum, cvals, image_original_adj, text_original_adj, item_emb)

    g = _sc_learned(em_d, idxs, cvs)

    ego = jnp.concatenate([user_emb, item_emb], axis=0)

    _prop_params = pltpu.CompilerParams(vmem_limit_bytes=120 * 1024 * 1024)

    e1 = pl.pallas_call(
        _prop1_body,
        grid=(N_UI_BLOCKS,),
        in_specs=[
            pl.BlockSpec((UIB, N_TOTAL), lambda i: (i, 0)),
            pl.BlockSpec((N_TOTAL, EMBED), lambda i: (0, 0)),
        ],
        out_specs=pl.BlockSpec((UIB, EMBED), lambda i: (i, 0)),
        out_shape=jax.ShapeDtypeStruct((N_TOTAL, EMBED), jnp.float32),
        compiler_params=_prop_params,
    )(adj, ego)

    all_e = pl.pallas_call(
        _prop2_body,
        grid=(N_UI_BLOCKS,),
        in_specs=[
            pl.BlockSpec((UIB, N_TOTAL), lambda i: (i, 0)),
            pl.BlockSpec((N_TOTAL, EMBED), lambda i: (0, 0)),
            pl.BlockSpec((N_TOTAL, EMBED), lambda i: (0, 0)),
            pl.BlockSpec((N_ITEMS, EMBED), lambda i: (0, 0)),
            pl.BlockSpec((N_ITEMS, EMBED), lambda i: (0, 0)),
        ],
        out_specs=pl.BlockSpec((UIB, EMBED), lambda i: (i, 0)),
        out_shape=jax.ShapeDtypeStruct((N_TOTAL, EMBED), jnp.float32),
        compiler_params=_prop_params,
    )(adj, ego, e1, h_dense, g)

    return all_e[:N_USERS], all_e[N_USERS:]


# speedup vs baseline: 1.0948x; 1.0948x over previous
import functools

import jax
import jax.numpy as jnp
from jax import lax
from jax.experimental import pallas as pl
from jax.experimental.pallas import tpu as pltpu
from jax.experimental.pallas import tpu_sc as plsc

N_USERS = 8192
N_ITEMS = 2048
EMBED = 64
TOPK = 10
NNZ = 2 * TOPK
LAMBDA = 0.9
N_TOTAL = N_USERS + N_ITEMS

PB = 256
N_P1 = N_TOTAL // PB
KNN_R = 64
KNN_STEPS = N_ITEMS // KNN_R
HB = 256
H_STEPS = N_ITEMS // HB
UIB = 512
N_UI_BLOCKS = N_TOTAL // UIB

SC_NC = 2
SC_NS = 16
SC_LANES = 16
SC_NW = SC_NC * SC_NS
SC_ROWS = N_ITEMS // SC_NW


def _feats_body(img_ref, wi_ref, bi_ref, txt_ref, wt_ref, bt_ref,
                xi_ref, xt_ref):
    fi = jnp.dot(img_ref[...], wi_ref[...],
                 preferred_element_type=jnp.float32) + bi_ref[...]
    ni = jnp.sqrt(jnp.sum(fi * fi, axis=1, keepdims=True))
    xi_ref[...] = fi / ni
    ft = jnp.dot(txt_ref[...], wt_ref[...],
                 preferred_element_type=jnp.float32) + bt_ref[...]
    nt = jnp.sqrt(jnp.sum(ft * ft, axis=1, keepdims=True))
    xt_ref[...] = ft / nt


def _topk_collect(sim, w, iota):
    BIG = jnp.float32(3.0e4)
    rows = sim.shape[0]
    vals, inds = [], []
    rowsum = jnp.zeros((rows, 1), jnp.float32)
    for _ in range(TOPK):
        m = jnp.max(sim, axis=1, keepdims=True)
        eq = sim == m
        idx = jnp.min(jnp.where(eq, iota, BIG), axis=1, keepdims=True)
        sel = iota == idx
        vals.append(w * m)
        inds.append(idx)
        rowsum = rowsum + w * m
        sim = jnp.where(sel, -jnp.inf, sim)
    ii = jnp.minimum(jnp.concatenate(inds, axis=1),
                     jnp.float32(N_ITEMS - 1)).astype(jnp.int32)
    return jnp.concatenate(vals, axis=1), ii, rowsum


def _p1_body(w_ref, adj_ref, ego_ref, xi_ref, xit_ref, xt_ref, xtt_ref,
             oi_ref, ot_ref, ie_ref,
             e1_ref, ix_ref, hd_ref, emd_ref, cvs_ref, rs_s, cv_s):
    pid = pl.program_id(0)
    e1_ref[...] = jnp.dot(adj_ref[...], ego_ref[...],
                          preferred_element_type=jnp.float32)

    @pl.when(pid < KNN_STEPS)
    def _():
        rows = pl.ds(pid * KNN_R, KNN_R)
        iota = jax.lax.broadcasted_iota(
            jnp.int32, (KNN_R, N_ITEMS), 1).astype(jnp.float32)
        sim_i = jnp.dot(xi_ref[rows, :], xit_ref[...],
                        preferred_element_type=jnp.float32)
        v0, i0, rs0 = _topk_collect(sim_i, w_ref[0], iota)
        sim_t = jnp.dot(xt_ref[rows, :], xtt_ref[...],
                        preferred_element_type=jnp.float32)
        v1, i1, rs1 = _topk_collect(sim_t, w_ref[1], iota)
        ix_ref[...] = jnp.concatenate([i0, i1], axis=1)
        cv_s[rows, :] = jnp.concatenate([v0, v1], axis=1)
        rs_s[rows, :] = rs0 + rs1

    @pl.when(pid >= KNN_STEPS)
    def _():
        hrows = pl.ds((pid - KNN_STEPS) * HB, HB)
        rsq_blk = jax.lax.rsqrt(rs_s[hrows, :])
        d_blk = jnp.where(jnp.isinf(rsq_blk), 0.0, rsq_blk)
        emd_ref[...] = d_blk * ie_ref[hrows, :]
        cvs_ref[...] = (1.0 - LAMBDA) * d_blk * cv_s[hrows, :]
        orig = w_ref[0] * oi_ref[...] + w_ref[1] * ot_ref[...]
        hd_ref[...] = LAMBDA * jnp.dot(orig, ie_ref[...],
                                       preferred_element_type=jnp.float32)


def _sc_learned_body(emd_hbm, ix_hbm, cvs_hbm, out_hbm,
                     ix_v, cvs_v, buf_a, buf_b, out_v, sem_a, sem_b):
    wid = lax.axis_index("s") * SC_NC + lax.axis_index("c")
    base = wid * SC_ROWS
    pltpu.sync_copy(ix_hbm.at[pl.ds(base, SC_ROWS)], ix_v)
    pltpu.sync_copy(cvs_hbm.at[pl.ds(base, SC_ROWS)], cvs_v)

    zero16 = jax.lax.iota(jnp.int32, 16) * 0

    def accum(r, buf):
        for c in range(EMBED // SC_LANES):
            acc = jnp.zeros((SC_LANES,), jnp.float32)
            for k in range(NNZ):
                cvb = plsc.load_gather(cvs_v, [zero16 + r, zero16 + k])
                acc = acc + cvb * buf[k, pl.ds(c * SC_LANES, SC_LANES)]
            out_v[r, pl.ds(c * SC_LANES, SC_LANES)] = acc

    cp0 = pltpu.async_copy(emd_hbm.at[ix_v.at[0]], buf_a, sem_a)

    def body(i, _):
        r = 2 * i
        cpb = pltpu.async_copy(emd_hbm.at[ix_v.at[r + 1]], buf_b, sem_b)
        pltpu.make_async_copy(emd_hbm.at[ix_v.at[r]], buf_a, sem_a).wait()
        accum(r, buf_a)
        cpa = pltpu.async_copy(emd_hbm.at[ix_v.at[(r + 2) % SC_ROWS]],
                               buf_a, sem_a)
        pltpu.make_async_copy(emd_hbm.at[ix_v.at[r + 1]], buf_b, sem_b).wait()
        accum(r + 1, buf_b)
        return 0

    lax.fori_loop(0, SC_ROWS // 2, body, 0)
    pltpu.make_async_copy(emd_hbm.at[ix_v.at[0]], buf_a, sem_a).wait()
    pltpu.sync_copy(out_v, out_hbm.at[pl.ds(base, SC_ROWS)])


@functools.partial(jax.jit, static_argnums=())
def _sc_learned(emd, ix, cvs):
    return pl.kernel(
        _sc_learned_body,
        out_type=jax.ShapeDtypeStruct((N_ITEMS, EMBED), jnp.float32),
        mesh=plsc.VectorSubcoreMesh(core_axis_name="c", subcore_axis_name="s"),
        compiler_params=pltpu.CompilerParams(needs_layout_passes=False,
                                             use_tc_tiling_on_sc=False),
        scratch_types=[
            pltpu.VMEM((SC_ROWS, NNZ), jnp.int32),
            pltpu.VMEM((SC_ROWS, NNZ), jnp.float32),
            pltpu.VMEM((NNZ, EMBED), jnp.float32),
            pltpu.VMEM((NNZ, EMBED), jnp.float32),
            pltpu.VMEM((SC_ROWS, EMBED), jnp.float32),
            pltpu.SemaphoreType.DMA,
            pltpu.SemaphoreType.DMA,
        ],
    )(emd, ix, cvs)


def _prop2_body(adj_ref, ego_ref, e1_ref, out_ref):
    pid = pl.program_id(0)
    rows = pl.ds(pid * UIB, UIB)
    e2 = jnp.dot(adj_ref[...], e1_ref[...], preferred_element_type=jnp.float32)
    out_ref[...] = (ego_ref[rows, :] + e1_ref[rows, :] + e2) * (1.0 / 3.0)


def _fin_body(ae_ref, hd_ref, g_ref, ig_ref):
    h = hd_ref[...] + g_ref[...]
    nrm = jnp.sqrt(jnp.sum(h * h, axis=1, keepdims=True))
    ig_ref[...] = ae_ref[...] + h / jnp.maximum(nrm, 1e-12)


def kernel(adj, user_emb, item_emb, image_emb, text_emb, W_img, b_img,
           W_txt, b_txt, modal_weight, image_original_adj, text_original_adj):
    w = jax.nn.softmax(modal_weight, axis=0)

    xn_img, xn_txt = pl.pallas_call(
        _feats_body,
        out_shape=(jax.ShapeDtypeStruct((N_ITEMS, EMBED), jnp.float32),
                   jax.ShapeDtypeStruct((N_ITEMS, EMBED), jnp.float32)),
    )(image_emb, W_img.astype(jnp.float32), b_img.reshape(1, EMBED),
      text_emb, W_txt.astype(jnp.float32), b_txt.reshape(1, EMBED))

    xn_img_t = xn_img.T
    xn_txt_t = xn_txt.T
    ego = jnp.concatenate([user_emb, item_emb], axis=0)

    def _h_map(i):
        return (jnp.minimum(jnp.maximum(i - KNN_STEPS, 0), H_STEPS - 1), 0)

    e1, idxs, h_dense, em_d, cvs = pl.pallas_call(
        _p1_body,
        grid=(N_P1,),
        in_specs=[
            pl.BlockSpec(memory_space=pltpu.SMEM),
            pl.BlockSpec((PB, N_TOTAL), lambda i: (i, 0)),
            pl.BlockSpec((N_TOTAL, EMBED), lambda i: (0, 0)),
            pl.BlockSpec((N_ITEMS, EMBED), lambda i: (0, 0)),
            pl.BlockSpec((EMBED, N_ITEMS), lambda i: (0, 0)),
            pl.BlockSpec((N_ITEMS, EMBED), lambda i: (0, 0)),
            pl.BlockSpec((EMBED, N_ITEMS), lambda i: (0, 0)),
            pl.BlockSpec((HB, N_ITEMS), _h_map),
            pl.BlockSpec((HB, N_ITEMS), _h_map),
            pl.BlockSpec((N_ITEMS, EMBED), lambda i: (0, 0)),
        ],
        out_specs=(
            pl.BlockSpec((PB, EMBED), lambda i: (i, 0)),
            pl.BlockSpec((KNN_R, NNZ),
                         lambda i: (jnp.minimum(i, KNN_STEPS - 1), 0)),
            pl.BlockSpec((HB, EMBED), _h_map),
            pl.BlockSpec((HB, EMBED), _h_map),
            pl.BlockSpec((HB, NNZ), _h_map),
        ),
        out_shape=(
            jax.ShapeDtypeStruct((N_TOTAL, EMBED), jnp.float32),
            jax.ShapeDtypeStruct((N_ITEMS, NNZ), jnp.int32),
            jax.ShapeDtypeStruct((N_ITEMS, EMBED), jnp.float32),
            jax.ShapeDtypeStruct((N_ITEMS, EMBED), jnp.float32),
            jax.ShapeDtypeStruct((N_ITEMS, NNZ), jnp.float32),
        ),
        scratch_shapes=[
            pltpu.VMEM((N_ITEMS, 1), jnp.float32),
            pltpu.VMEM((N_ITEMS, NNZ), jnp.float32),
        ],
    )(w, adj, ego, xn_img, xn_img_t, xn_txt, xn_txt_t,
      image_original_adj, text_original_adj, item_emb)

    g = _sc_learned(em_d, idxs, cvs)

    all_e = pl.pallas_call(
        _prop2_body,
        grid=(N_UI_BLOCKS,),
        in_specs=[
            pl.BlockSpec((UIB, N_TOTAL), lambda i: (i, 0)),
            pl.BlockSpec((N_TOTAL, EMBED), lambda i: (0, 0)),
            pl.BlockSpec((N_TOTAL, EMBED), lambda i: (0, 0)),
        ],
        out_specs=pl.BlockSpec((UIB, EMBED), lambda i: (i, 0)),
        out_shape=jax.ShapeDtypeStruct((N_TOTAL, EMBED), jnp.float32),
    )(adj, ego, e1)

    i_g = pl.pallas_call(
        _fin_body,
        grid=(1,),
        in_specs=[
            pl.BlockSpec((N_ITEMS, EMBED), lambda i: (N_USERS // N_ITEMS, 0)),
            pl.BlockSpec((N_ITEMS, EMBED), lambda i: (0, 0)),
            pl.BlockSpec((N_ITEMS, EMBED), lambda i: (0, 0)),
        ],
        out_specs=pl.BlockSpec((N_ITEMS, EMBED), lambda i: (0, 0)),
        out_shape=jax.ShapeDtypeStruct((N_ITEMS, EMBED), jnp.float32),
    )(all_e, h_dense, g)

    return all_e[:N_USERS], i_g


# speedup vs baseline: 1.1078x; 1.0119x over previous
import functools

import jax
import jax.numpy as jnp
from jax import lax
from jax.experimental import pallas as pl
from jax.experimental.pallas import tpu as pltpu
from jax.experimental.pallas import tpu_sc as plsc

N_USERS = 8192
N_ITEMS = 2048
EMBED = 64
TOPK = 10
NNZ = 2 * TOPK
LAMBDA = 0.9
N_TOTAL = N_USERS + N_ITEMS

PB = 256
N_P1 = N_TOTAL // PB
KNN_R = 64
KNN_STEPS = N_ITEMS // KNN_R
HB = 256
H_STEPS = N_ITEMS // HB
UIB = 512
N_UI_BLOCKS = N_TOTAL // UIB

SC_NC = 2
SC_NS = 16
SC_LANES = 16
SC_NW = SC_NC * SC_NS
SC_ROWS = N_ITEMS // SC_NW


def _feats_body(img_ref, wi_ref, bi_ref, txt_ref, wt_ref, bt_ref,
                xi_ref, xt_ref):
    fi = jnp.dot(img_ref[...], wi_ref[...],
                 preferred_element_type=jnp.float32) + bi_ref[...]
    ni = jnp.sqrt(jnp.sum(fi * fi, axis=1, keepdims=True))
    xi_ref[...] = fi / ni
    ft = jnp.dot(txt_ref[...], wt_ref[...],
                 preferred_element_type=jnp.float32) + bt_ref[...]
    nt = jnp.sqrt(jnp.sum(ft * ft, axis=1, keepdims=True))
    xt_ref[...] = ft / nt


def _topk_collect(sim, w, iota):
    BIG = jnp.float32(3.0e4)
    rows = sim.shape[0]
    vals, inds = [], []
    rowsum = jnp.zeros((rows, 1), jnp.float32)
    for _ in range(TOPK):
        m = jnp.max(sim, axis=1, keepdims=True)
        eq = sim == m
        idx = jnp.min(jnp.where(eq, iota, BIG), axis=1, keepdims=True)
        sel = iota == idx
        vals.append(w * m)
        inds.append(idx)
        rowsum = rowsum + w * m
        sim = jnp.where(sel, -jnp.inf, sim)
    ii = jnp.minimum(jnp.concatenate(inds, axis=1),
                     jnp.float32(N_ITEMS - 1)).astype(jnp.int32)
    return jnp.concatenate(vals, axis=1), ii, rowsum


def _p1_body(w_ref, adj_ref, ego_ref, xi_ref, xit_ref, xt_ref, xtt_ref,
             oi_ref, ot_ref, ie_ref,
             e1_ref, ix_ref, hd_ref, emd_ref, cvs_ref, rs_s, cv_s):
    pid = pl.program_id(0)
    e1_ref[...] = jnp.dot(adj_ref[...], ego_ref[...],
                          preferred_element_type=jnp.float32)

    @pl.when(pid < KNN_STEPS)
    def _():
        rows = pl.ds(pid * KNN_R, KNN_R)
        iota = jax.lax.broadcasted_iota(
            jnp.int32, (KNN_R, N_ITEMS), 1).astype(jnp.float32)
        sim_i = jnp.dot(xi_ref[rows, :], xit_ref[...],
                        preferred_element_type=jnp.float32)
        v0, i0, rs0 = _topk_collect(sim_i, w_ref[0], iota)
        sim_t = jnp.dot(xt_ref[rows, :], xtt_ref[...],
                        preferred_element_type=jnp.float32)
        v1, i1, rs1 = _topk_collect(sim_t, w_ref[1], iota)
        ix_ref[...] = jnp.concatenate([i0, i1], axis=1)
        cv_s[rows, :] = jnp.concatenate([v0, v1], axis=1)
        rs_s[rows, :] = rs0 + rs1

    @pl.when(pid >= KNN_STEPS)
    def _():
        hrows = pl.ds((pid - KNN_STEPS) * HB, HB)
        rsq_blk = jax.lax.rsqrt(rs_s[hrows, :])
        d_blk = jnp.where(jnp.isinf(rsq_blk), 0.0, rsq_blk)
        emd_ref[...] = d_blk * ie_ref[hrows, :]
        cvs_ref[...] = (1.0 - LAMBDA) * d_blk * cv_s[hrows, :]
        orig = w_ref[0] * oi_ref[...] + w_ref[1] * ot_ref[...]
        hd_ref[...] = LAMBDA * jnp.dot(orig, ie_ref[...],
                                       preferred_element_type=jnp.float32)


def _sc_learned_body(emd_hbm, ix_hbm, cvs_hbm, out_hbm,
                     ix_v, cvs_v, buf_a, buf_b, out_v, sem_a, sem_b):
    wid = lax.axis_index("s") * SC_NC + lax.axis_index("c")
    base = wid * SC_ROWS
    pltpu.sync_copy(ix_hbm.at[pl.ds(base, SC_ROWS)], ix_v)
    pltpu.sync_copy(cvs_hbm.at[pl.ds(base, SC_ROWS)], cvs_v)

    zero16 = jax.lax.iota(jnp.int32, 16) * 0

    def accum(r, buf):
        for c in range(EMBED // SC_LANES):
            acc = jnp.zeros((SC_LANES,), jnp.float32)
            for k in range(NNZ):
                cvb = plsc.load_gather(cvs_v, [zero16 + r, zero16 + k])
                acc = acc + cvb * buf[k, pl.ds(c * SC_LANES, SC_LANES)]
            out_v[r, pl.ds(c * SC_LANES, SC_LANES)] = acc

    cp0 = pltpu.async_copy(emd_hbm.at[ix_v.at[0]], buf_a, sem_a)

    def body(i, _):
        r = 2 * i
        cpb = pltpu.async_copy(emd_hbm.at[ix_v.at[r + 1]], buf_b, sem_b)
        pltpu.make_async_copy(emd_hbm.at[ix_v.at[r]], buf_a, sem_a).wait()
        accum(r, buf_a)
        cpa = pltpu.async_copy(emd_hbm.at[ix_v.at[(r + 2) % SC_ROWS]],
                               buf_a, sem_a)
        pltpu.make_async_copy(emd_hbm.at[ix_v.at[r + 1]], buf_b, sem_b).wait()
        accum(r + 1, buf_b)
        return 0

    lax.fori_loop(0, SC_ROWS // 2, body, 0)
    pltpu.make_async_copy(emd_hbm.at[ix_v.at[0]], buf_a, sem_a).wait()
    pltpu.sync_copy(out_v, out_hbm.at[pl.ds(base, SC_ROWS)])


@functools.partial(jax.jit, static_argnums=())
def _sc_learned(emd, ix, cvs):
    return pl.kernel(
        _sc_learned_body,
        out_type=jax.ShapeDtypeStruct((N_ITEMS, EMBED), jnp.float32),
        mesh=plsc.VectorSubcoreMesh(core_axis_name="c", subcore_axis_name="s"),
        compiler_params=pltpu.CompilerParams(needs_layout_passes=False,
                                             use_tc_tiling_on_sc=False),
        scratch_types=[
            pltpu.VMEM((SC_ROWS, NNZ), jnp.int32),
            pltpu.VMEM((SC_ROWS, NNZ), jnp.float32),
            pltpu.VMEM((NNZ, EMBED), jnp.float32),
            pltpu.VMEM((NNZ, EMBED), jnp.float32),
            pltpu.VMEM((SC_ROWS, EMBED), jnp.float32),
            pltpu.SemaphoreType.DMA,
            pltpu.SemaphoreType.DMA,
        ],
    )(emd, ix, cvs)


def _prop2_body(adj_ref, ego_ref, e1_ref, ug_ref, ai_ref):
    pid = pl.program_id(0)
    rows = pl.ds(pid * UIB, UIB)
    e2 = jnp.dot(adj_ref[...], e1_ref[...], preferred_element_type=jnp.float32)
    acc = (ego_ref[rows, :] + e1_ref[rows, :] + e2) * (1.0 / 3.0)

    @pl.when(pid < N_USERS // UIB)
    def _():
        ug_ref[...] = acc

    @pl.when(pid >= N_USERS // UIB)
    def _():
        ai_ref[...] = acc


def _fin_body(ae_ref, hd_ref, g_ref, ig_ref):
    h = hd_ref[...] + g_ref[...]
    nrm = jnp.sqrt(jnp.sum(h * h, axis=1, keepdims=True))
    ig_ref[...] = ae_ref[...] + h / jnp.maximum(nrm, 1e-12)


def kernel(adj, user_emb, item_emb, image_emb, text_emb, W_img, b_img,
           W_txt, b_txt, modal_weight, image_original_adj, text_original_adj):
    w = jax.nn.softmax(modal_weight, axis=0)

    xn_img, xn_txt = pl.pallas_call(
        _feats_body,
        out_shape=(jax.ShapeDtypeStruct((N_ITEMS, EMBED), jnp.float32),
                   jax.ShapeDtypeStruct((N_ITEMS, EMBED), jnp.float32)),
    )(image_emb, W_img.astype(jnp.float32), b_img.reshape(1, EMBED),
      text_emb, W_txt.astype(jnp.float32), b_txt.reshape(1, EMBED))

    xn_img_t = xn_img.T
    xn_txt_t = xn_txt.T
    ego = jnp.concatenate([user_emb, item_emb], axis=0)

    def _h_map(i):
        return (jnp.minimum(jnp.maximum(i - KNN_STEPS, 0), H_STEPS - 1), 0)

    e1, idxs, h_dense, em_d, cvs = pl.pallas_call(
        _p1_body,
        grid=(N_P1,),
        in_specs=[
            pl.BlockSpec(memory_space=pltpu.SMEM),
            pl.BlockSpec((PB, N_TOTAL), lambda i: (i, 0)),
            pl.BlockSpec((N_TOTAL, EMBED), lambda i: (0, 0)),
            pl.BlockSpec((N_ITEMS, EMBED), lambda i: (0, 0)),
            pl.BlockSpec((EMBED, N_ITEMS), lambda i: (0, 0)),
            pl.BlockSpec((N_ITEMS, EMBED), lambda i: (0, 0)),
            pl.BlockSpec((EMBED, N_ITEMS), lambda i: (0, 0)),
            pl.BlockSpec((HB, N_ITEMS), _h_map),
            pl.BlockSpec((HB, N_ITEMS), _h_map),
            pl.BlockSpec((N_ITEMS, EMBED), lambda i: (0, 0)),
        ],
        out_specs=(
            pl.BlockSpec((PB, EMBED), lambda i: (i, 0)),
            pl.BlockSpec((KNN_R, NNZ),
                         lambda i: (jnp.minimum(i, KNN_STEPS - 1), 0)),
            pl.BlockSpec((HB, EMBED), _h_map),
            pl.BlockSpec((HB, EMBED), _h_map),
            pl.BlockSpec((HB, NNZ), _h_map),
        ),
        out_shape=(
            jax.ShapeDtypeStruct((N_TOTAL, EMBED), jnp.float32),
            jax.ShapeDtypeStruct((N_ITEMS, NNZ), jnp.int32),
            jax.ShapeDtypeStruct((N_ITEMS, EMBED), jnp.float32),
            jax.ShapeDtypeStruct((N_ITEMS, EMBED), jnp.float32),
            jax.ShapeDtypeStruct((N_ITEMS, NNZ), jnp.float32),
        ),
        scratch_shapes=[
            pltpu.VMEM((N_ITEMS, 1), jnp.float32),
            pltpu.VMEM((N_ITEMS, NNZ), jnp.float32),
        ],
    )(w, adj, ego, xn_img, xn_img_t, xn_txt, xn_txt_t,
      image_original_adj, text_original_adj, item_emb)

    g = _sc_learned(em_d, idxs, cvs)

    n_ub = N_USERS // UIB
    u_g, ae_items = pl.pallas_call(
        _prop2_body,
        grid=(N_UI_BLOCKS,),
        in_specs=[
            pl.BlockSpec((UIB, N_TOTAL), lambda i: (i, 0)),
            pl.BlockSpec((N_TOTAL, EMBED), lambda i: (0, 0)),
            pl.BlockSpec((N_TOTAL, EMBED), lambda i: (0, 0)),
        ],
        out_specs=(
            pl.BlockSpec((UIB, EMBED),
                         lambda i: (jnp.minimum(i, n_ub - 1), 0)),
            pl.BlockSpec((UIB, EMBED),
                         lambda i: (jnp.maximum(i - n_ub, 0), 0)),
        ),
        out_shape=(jax.ShapeDtypeStruct((N_USERS, EMBED), jnp.float32),
                   jax.ShapeDtypeStruct((N_ITEMS, EMBED), jnp.float32)),
    )(adj, ego, e1)

    i_g = pl.pallas_call(
        _fin_body,
        grid=(1,),
        in_specs=[
            pl.BlockSpec((N_ITEMS, EMBED), lambda i: (0, 0)),
            pl.BlockSpec((N_ITEMS, EMBED), lambda i: (0, 0)),
            pl.BlockSpec((N_ITEMS, EMBED), lambda i: (0, 0)),
        ],
        out_specs=pl.BlockSpec((N_ITEMS, EMBED), lambda i: (0, 0)),
        out_shape=jax.ShapeDtypeStruct((N_ITEMS, EMBED), jnp.float32),
    )(ae_items, h_dense, g)

    return u_g, i_g
